# baseline (speedup 1.0000x reference)
import functools

import jax
import jax.numpy as jnp
from jax import lax
from jax.experimental import pallas as pl
from jax.experimental.pallas import tpu as pltpu
from jax.experimental.pallas import tpu_sc as plsc

N = 10000
F = 128
C = 64
WD = 128
G = 8
LIN = 128
OUT = 10
E = 320000
ETOT = E + N

NC = 2
NS = 16
NW = NC * NS
CHUNK = 128
NCHUNK = 82
NPAIR = NCHUNK // 2
EPW = NCHUNK * CHUNK
ETOT_PAD = EPW * NW
N_PAD = 10240
ROWS_PT = N_PAD // NS


def _tc_proj(x, w, attd):
    def body(x_ref, w_ref, attd_ref, h_ref, ad_ref, ms_ref):
        h = jnp.dot(x_ref[...], w_ref[...], preferred_element_type=jnp.float32)
        a_d = jnp.sum(h * attd_ref[...], axis=1)
        h_ref[...] = h
        ad_ref[...] = a_d
        ms_ref[...] = jnp.broadcast_to(jnp.max(h[:, C + 1:C + 2]), (128,))

    return pl.pallas_call(
        body,
        out_shape=[
            jax.ShapeDtypeStruct((N, WD), jnp.float32),
            jax.ShapeDtypeStruct((N,), jnp.float32),
            jax.ShapeDtypeStruct((128,), jnp.float32),
        ],
    )(x, w, attd)


def _tc_combine_proj(parts, bias, w, attd):
    def body(p_ref, b_ref, w_ref, attd_ref,
             h_ref, ad_ref, ms_ref):
        comb = (p_ref[0] + p_ref[1])[:N]
        den = comb[:, C:C + 1] + 1e-16
        o = comb[:, :C] / den + b_ref[...]
        x2 = jnp.maximum(o, 0.0)
        h = jnp.dot(x2, w_ref[...], preferred_element_type=jnp.float32)
        a_d = jnp.sum(h * attd_ref[...], axis=1)
        h_ref[...] = h
        ad_ref[...] = a_d
        ms_ref[...] = jnp.broadcast_to(jnp.max(h[:, C + 1:C + 2]), (128,))

    return pl.pallas_call(
        body,
        out_shape=[
            jax.ShapeDtypeStruct((N, WD), jnp.float32),
            jax.ShapeDtypeStruct((N,), jnp.float32),
            jax.ShapeDtypeStruct((128,), jnp.float32),
        ],
    )(parts, bias, w, attd)


def _tc_head(parts, bias, batch, linW, linb, outW, outb):
    def body(p_ref, b_ref, batch_ref, lw_ref, lb_ref, ow_ref, ob_ref,
             out_ref):
        comb = (p_ref[0] + p_ref[1])[:N]
        den = comb[:, C:C + 1] + 1e-16
        o = comb[:, :C] / den + b_ref[...]
        o = jnp.maximum(o, 0.0)
        b = batch_ref[...]
        rows = []
        for g in range(G):
            m = (b == g)
            rows.append(jnp.max(jnp.where(m, o, -jnp.inf), axis=0,
                                keepdims=True))
        gm = jnp.concatenate(rows, axis=0)
        g1 = jnp.dot(gm, lw_ref[...], preferred_element_type=jnp.float32)
        g1 = g1 + lb_ref[...]
        out = jnp.dot(g1, ow_ref[...], preferred_element_type=jnp.float32)
        out_ref[...] = out + ob_ref[...]

    return pl.pallas_call(
        body,
        out_shape=jax.ShapeDtypeStruct((G, OUT), jnp.float32),
    )(parts, bias, batch, linW, linb, outW, outb)


def _sc_edge_pass(h, a_dst, msvec, src4, dst4, z_rows):
    mesh = plsc.VectorSubcoreMesh(core_axis_name="c", subcore_axis_name="s")

    @functools.partial(
        pl.kernel,
        out_type=jax.ShapeDtypeStruct((NC, N_PAD, WD), jnp.float32),
        mesh=mesh,
        compiler_params=pltpu.CompilerParams(needs_layout_passes=False),
        scratch_types=[
            pltpu.VMEM((2, 2, CHUNK), jnp.int32),
            pltpu.VMEM((2, 2, CHUNK), jnp.int32),
            pltpu.VMEM((2, CHUNK), jnp.int32),
            pltpu.VMEM((N,), jnp.float32),
            pltpu.VMEM((16,), jnp.float32),
            pltpu.VMEM((2, CHUNK, WD), jnp.float32),
            pltpu.MemorySpace.VMEM_SHARED((N_PAD, WD), jnp.float32),
            pltpu.SemaphoreType.DMA,
            pltpu.SemaphoreType.DMA,
            pltpu.SemaphoreType.DMA,
            pltpu.SemaphoreType.DMA,
            pltpu.SemaphoreType.DMA,
            pltpu.SemaphoreType.DMA,
        ],
    )
    def k(h_hbm, ad_hbm, ms_hbm, src_hbm, dst_hbm, zr_hbm,
          outp_hbm,
          srcp_v, dstp_v, dstsc_v, ad_v, ms_v, rows_v, acc_sh,
          semi0, semi1, semr0, semr1, sems0, sems1):
        semi = (semi0, semi1)
        semr = (semr0, semr1)
        sems = (sems0, sems1)
        c = lax.axis_index("c")
        s = lax.axis_index("s")
        wid = s * NC + c
        pltpu.sync_copy(ad_hbm, ad_v)
        pltpu.sync_copy(ms_hbm.at[pl.ds(0, 16)], ms_v)
        pltpu.sync_copy(zr_hbm.at[pl.ds(s * ROWS_PT, ROWS_PT)],
                        acc_sh.at[pl.ds(s * ROWS_PT, ROWS_PT)])
        plsc.subcore_barrier()

        ebase = wid * EPW
        iota16 = lax.iota(jnp.int32, 16)
        col64 = jnp.full((16,), C, jnp.int32)
        col65 = jnp.full((16,), C + 1, jnp.int32)

        def fetch_pair(p, slot):
            pltpu.async_copy(src_hbm.at[wid, p], srcp_v.at[slot], semi[slot])
            pltpu.async_copy(dst_hbm.at[wid, p], dstp_v.at[slot], semi[slot])

        def wait_pair(slot):
            pltpu.make_async_copy(src_hbm.at[wid, 0], srcp_v.at[slot],
                                  semi[slot]).wait()
            pltpu.make_async_copy(dst_hbm.at[wid, 0], dstp_v.at[slot],
                                  semi[slot]).wait()

        def start_gather(slot, rb):
            pltpu.async_copy(h_hbm.at[srcp_v.at[slot, rb]], rows_v.at[rb],
                             semr[rb])

        def wait_gather(rb):
            pltpu.make_async_copy(h_hbm.at[srcp_v.at[0, rb]], rows_v.at[rb],
                                  semr[rb]).wait()

        def start_scatter(rb):
            pltpu.async_copy(rows_v.at[rb], acc_sh.at[dstsc_v.at[rb]],
                             sems[rb], add=True)

        def wait_scatter(rb):
            pltpu.make_async_copy(rows_v.at[rb], acc_sh.at[dstsc_v.at[rb]],
                                  sems[rb]).wait()

        def compute_chunk(j, slot, rb):
            ms16 = ms_v[...]
            rows_rb = rows_v.at[rb]
            for o in range(CHUNK // 16):
                rowg = o * 16 + iota16
                a_s = plsc.load_gather(rows_rb, [rowg, col65])
                dstg = dstp_v[slot, rb, pl.ds(o * 16, 16)]
                a_d = plsc.load_gather(ad_v, [dstg])
                kk = ms16 + a_d
                kk = jnp.where(kk >= 0, kk, 0.2 * kk)
                al = a_s + a_d
                al = jnp.where(al >= 0, al, 0.2 * al)
                al = jnp.exp(al - kk)
                pos = ebase + j * CHUNK + o * 16 + iota16
                al = jnp.where(pos < ETOT, al, 0.0)
                plsc.store_scatter(rows_rb, [rowg, col64], al)
                for e in range(16):
                    a = al[e]
                    row = o * 16 + e
                    for cg in range(C // 16):
                        sl = pl.ds(cg * 16, 16)
                        rows_v[rb, row, sl] = rows_v[rb, row, sl] * a
                dstsc_v[rb, pl.ds(o * 16, 16)] = dstg

        pltpu.sync_copy(src_hbm.at[wid, 0], srcp_v.at[0])
        pltpu.sync_copy(dst_hbm.at[wid, 0], dstp_v.at[0])
        start_gather(0, 0)
        fetch_pair(1, 1)
        start_gather(0, 1)
        wait_gather(0)
        compute_chunk(0, 0, 0)
        start_scatter(0)
        wait_gather(1)
        compute_chunk(1, 0, 1)
        wait_pair(1)
        wait_scatter(0)
        start_gather(1, 0)
        start_scatter(1)

        def pair_steps(p, slot):
            j0 = 2 * p
            j1 = j0 + 1
            fetch_pair(p + 1, 1 - slot)
            wait_scatter(1)
            start_gather(slot, 1)
            wait_gather(0)
            compute_chunk(j0, slot, 0)
            start_scatter(0)
            wait_gather(1)
            compute_chunk(j1, slot, 1)
            wait_pair(1 - slot)
            wait_scatter(0)
            start_gather(1 - slot, 0)
            start_scatter(1)

        def body2(i, carry):
            pair_steps(1 + 2 * i, 1)
            pair_steps(2 + 2 * i, 0)
            return carry

        lax.fori_loop(0, (NPAIR - 1) // 2, body2, 0)

        wait_gather(0)
        wait_scatter(1)
        plsc.subcore_barrier()
        pltpu.sync_copy(acc_sh.at[pl.ds(s * ROWS_PT, ROWS_PT)],
                        outp_hbm.at[c, pl.ds(s * ROWS_PT, ROWS_PT)])

    return k(h, a_dst, msvec, src4, dst4, z_rows)


def kernel(x, edge_index, batch, W1, att_src1, att_dst1, b1,
           W2, att_src2, att_dst2, b2, linW, linb, outW, outb):
    loop = jnp.arange(N, dtype=edge_index.dtype)
    src = jnp.concatenate([edge_index[0], loop])
    dst = jnp.concatenate([edge_index[1], loop])
    pad = ETOT_PAD - ETOT
    zpad = jnp.zeros((pad,), dtype=src.dtype)
    src4 = jnp.concatenate([src, zpad]).reshape(NW, NPAIR, 2, CHUNK)
    dst4 = jnp.concatenate([dst, zpad]).reshape(NW, NPAIR, 2, CHUNK)
    src4 = jnp.pad(src4, ((0, 0), (0, 1), (0, 0), (0, 0)))
    dst4 = jnp.pad(dst4, ((0, 0), (0, 1), (0, 0), (0, 0)))
    z_rows = jnp.zeros((N_PAD, WD), jnp.float32)

    def aug_w(wmat, att_s):
        acol = wmat @ att_s.reshape(C, 1)
        zcol = jnp.zeros_like(acol)
        tail = jnp.zeros((wmat.shape[0], WD - C - 2), wmat.dtype)
        return jnp.concatenate([wmat, zcol, acol, tail], axis=1)

    W1p = aug_w(W1, att_src1)
    W2p = aug_w(W2, att_src2)
    attd1 = jnp.pad(att_dst1.reshape(1, C), ((0, 0), (0, WD - C)))
    attd2 = jnp.pad(att_dst2.reshape(1, C), ((0, 0), (0, WD - C)))
    batch2 = batch.reshape(N, 1)
    b1r = b1.reshape(1, C)
    b2r = b2.reshape(1, C)
    linbr = linb.reshape(1, LIN)
    outbr = outb.reshape(1, OUT)

    h1, ad1, ms1 = _tc_proj(x, W1p, attd1)
    p1 = _sc_edge_pass(h1, ad1, ms1, src4, dst4, z_rows)
    h2, ad2, ms2 = _tc_combine_proj(p1, b1r, W2p, attd2)
    p2 = _sc_edge_pass(h2, ad2, ms2, src4, dst4, z_rows)
    return _tc_head(p2, b2r, batch2, linW, linbr, outW, outbr)


# speedup vs baseline: 1.2377x; 1.2377x over previous
import functools

import jax
import jax.numpy as jnp
from jax import lax
from jax.experimental import pallas as pl
from jax.experimental.pallas import tpu as pltpu
from jax.experimental.pallas import tpu_sc as plsc

N = 10000
F = 128
C = 64
WD = 128
G = 8
LIN = 128
OUT = 10
E = 320000
ETOT = E + N

NC = 2
NS = 16
NW = NC * NS
CHUNK = 128
NCHUNK = 82
NPAIR = NCHUNK // 2
EPW = NCHUNK * CHUNK
ETOT_PAD = EPW * NW
N_PAD = 10240
ROWS_PT = N_PAD // NS


def _tc_proj(x, w, attd):
    def body(x_ref, w_ref, attd_ref, h_ref, ad_ref, ms_ref):
        h = jnp.dot(x_ref[...], w_ref[...], preferred_element_type=jnp.float32)
        a_d = jnp.sum(h * attd_ref[...], axis=1)
        h_ref[...] = h
        ad_ref[...] = a_d
        ms_ref[...] = jnp.broadcast_to(jnp.max(h[:, C + 1:C + 2]), (128,))

    return pl.pallas_call(
        body,
        out_shape=[
            jax.ShapeDtypeStruct((N, WD), jnp.float32),
            jax.ShapeDtypeStruct((N,), jnp.float32),
            jax.ShapeDtypeStruct((128,), jnp.float32),
        ],
    )(x, w, attd)


def _tc_combine_proj(parts, bias, w, attd):
    def body(p_ref, b_ref, w_ref, attd_ref,
             h_ref, ad_ref, ms_ref):
        comb = (p_ref[0] + p_ref[1])[:N]
        den = comb[:, C:C + 1] + 1e-16
        o = comb[:, :C] / den + b_ref[...]
        x2 = jnp.maximum(o, 0.0)
        h = jnp.dot(x2, w_ref[...], preferred_element_type=jnp.float32)
        a_d = jnp.sum(h * attd_ref[...], axis=1)
        h_ref[...] = h
        ad_ref[...] = a_d
        ms_ref[...] = jnp.broadcast_to(jnp.max(h[:, C + 1:C + 2]), (128,))

    return pl.pallas_call(
        body,
        out_shape=[
            jax.ShapeDtypeStruct((N, WD), jnp.float32),
            jax.ShapeDtypeStruct((N,), jnp.float32),
            jax.ShapeDtypeStruct((128,), jnp.float32),
        ],
    )(parts, bias, w, attd)


def _tc_head(parts, bias, batch, linW, linb, outW, outb):
    def body(p_ref, b_ref, batch_ref, lw_ref, lb_ref, ow_ref, ob_ref,
             out_ref):
        comb = (p_ref[0] + p_ref[1])[:N]
        den = comb[:, C:C + 1] + 1e-16
        o = comb[:, :C] / den + b_ref[...]
        o = jnp.maximum(o, 0.0)
        b = batch_ref[...]
        rows = []
        for g in range(G):
            m = (b == g)
            rows.append(jnp.max(jnp.where(m, o, -jnp.inf), axis=0,
                                keepdims=True))
        gm = jnp.concatenate(rows, axis=0)
        g1 = jnp.dot(gm, lw_ref[...], preferred_element_type=jnp.float32)
        g1 = g1 + lb_ref[...]
        out = jnp.dot(g1, ow_ref[...], preferred_element_type=jnp.float32)
        out_ref[...] = out + ob_ref[...]

    return pl.pallas_call(
        body,
        out_shape=jax.ShapeDtypeStruct((G, OUT), jnp.float32),
    )(parts, bias, batch, linW, linb, outW, outb)


def _sc_edge_pass(h, a_dst, msvec, src4, dst4, z_rows):
    mesh = plsc.VectorSubcoreMesh(core_axis_name="c", subcore_axis_name="s")

    @functools.partial(
        pl.kernel,
        out_type=jax.ShapeDtypeStruct((NC, N_PAD, WD), jnp.float32),
        mesh=mesh,
        compiler_params=pltpu.CompilerParams(needs_layout_passes=False),
        scratch_types=[
            pltpu.VMEM((2, 2, CHUNK), jnp.int32),
            pltpu.VMEM((2, 2, CHUNK), jnp.int32),
            pltpu.VMEM((2, CHUNK), jnp.int32),
            pltpu.VMEM((N,), jnp.float32),
            pltpu.VMEM((16,), jnp.float32),
            pltpu.VMEM((2, CHUNK, WD), jnp.float32),
            pltpu.MemorySpace.VMEM_SHARED((N_PAD, WD), jnp.float32),
            pltpu.SemaphoreType.DMA,
            pltpu.SemaphoreType.DMA,
            pltpu.SemaphoreType.DMA,
            pltpu.SemaphoreType.DMA,
            pltpu.SemaphoreType.DMA,
            pltpu.SemaphoreType.DMA,
        ],
    )
    def k(h_hbm, ad_hbm, ms_hbm, src_hbm, dst_hbm, zr_hbm,
          outp_hbm,
          srcp_v, dstp_v, dstsc_v, ad_v, ms_v, rows_v, acc_sh,
          semi0, semi1, semr0, semr1, sems0, sems1):
        semi = (semi0, semi1)
        semr = (semr0, semr1)
        sems = (sems0, sems1)
        c = lax.axis_index("c")
        s = lax.axis_index("s")
        wid = s * NC + c
        pltpu.sync_copy(ad_hbm, ad_v)
        pltpu.sync_copy(ms_hbm.at[pl.ds(0, 16)], ms_v)
        pltpu.sync_copy(zr_hbm.at[pl.ds(s * ROWS_PT, ROWS_PT)],
                        acc_sh.at[pl.ds(s * ROWS_PT, ROWS_PT)])
        plsc.subcore_barrier()

        ebase = wid * EPW
        iota16 = lax.iota(jnp.int32, 16)
        col64 = jnp.full((16,), C, jnp.int32)
        col65 = jnp.full((16,), C + 1, jnp.int32)

        def fetch_pair(p, slot):
            pltpu.async_copy(src_hbm.at[wid, p], srcp_v.at[slot], semi[slot])
            pltpu.async_copy(dst_hbm.at[wid, p], dstp_v.at[slot], semi[slot])

        def wait_pair(slot):
            pltpu.make_async_copy(src_hbm.at[wid, 0], srcp_v.at[slot],
                                  semi[slot]).wait()
            pltpu.make_async_copy(dst_hbm.at[wid, 0], dstp_v.at[slot],
                                  semi[slot]).wait()

        def start_gather(slot, rb):
            pltpu.async_copy(h_hbm.at[srcp_v.at[slot, rb]], rows_v.at[rb],
                             semr[rb])

        def wait_gather(rb):
            pltpu.make_async_copy(h_hbm.at[srcp_v.at[0, rb]], rows_v.at[rb],
                                  semr[rb]).wait()

        def start_gather2(slot):
            pltpu.async_copy(h_hbm.at[srcp_v.at[slot, 1]], rows_v.at[1],
                             semr[1])

        def wait_gather2():
            pltpu.make_async_copy(h_hbm.at[srcp_v.at[0, 1]], rows_v.at[1],
                                  semr[1]).wait()

        def start_scatter(rb):
            pltpu.async_copy(rows_v.at[rb], acc_sh.at[dstsc_v.at[rb]],
                             sems[rb], add=True)

        def wait_scatter(rb):
            pltpu.make_async_copy(rows_v.at[rb], acc_sh.at[dstsc_v.at[rb]],
                                  sems[rb]).wait()

        def compute_chunk(j, slot, rb):
            ms16 = ms_v[...]
            rows_rb = rows_v.at[rb]
            for o in range(CHUNK // 16):
                rowg = o * 16 + iota16
                a_s = plsc.load_gather(rows_rb, [rowg, col65])
                dstg = dstp_v[slot, rb, pl.ds(o * 16, 16)]
                a_d = plsc.load_gather(ad_v, [dstg])
                kk = ms16 + a_d
                kk = jnp.where(kk >= 0, kk, 0.2 * kk)
                al = a_s + a_d
                al = jnp.where(al >= 0, al, 0.2 * al)
                al = jnp.exp(al - kk)
                pos = ebase + j * CHUNK + o * 16 + iota16
                al = jnp.where(pos < ETOT, al, 0.0)
                plsc.store_scatter(rows_rb, [rowg, col64], al)
                for e in range(16):
                    a = al[e]
                    row = o * 16 + e
                    for cg in range(C // 16):
                        sl = pl.ds(cg * 16, 16)
                        rows_v[rb, row, sl] = rows_v[rb, row, sl] * a
                dstsc_v[rb, pl.ds(o * 16, 16)] = dstg

        def pair_body(p, carry):
            pltpu.sync_copy(src_hbm.at[wid, p], srcp_v.at[0])
            pltpu.sync_copy(dst_hbm.at[wid, p], dstp_v.at[0])

            @pl.when(p > 0)
            def _():
                wait_scatter(0)
            start_gather(0, 0)

            @pl.when(p > 0)
            def _():
                wait_scatter(1)
            start_gather2(0)

            wait_gather(0)
            compute_chunk(2 * p, 0, 0)
            start_scatter(0)
            wait_gather2()
            compute_chunk(2 * p + 1, 0, 1)
            start_scatter(1)
            return carry

        lax.fori_loop(0, NPAIR, pair_body, 0)
        wait_scatter(0)
        wait_scatter(1)
        plsc.subcore_barrier()
        pltpu.sync_copy(acc_sh.at[pl.ds(s * ROWS_PT, ROWS_PT)],
                        outp_hbm.at[c, pl.ds(s * ROWS_PT, ROWS_PT)])

    return k(h, a_dst, msvec, src4, dst4, z_rows)


def kernel(x, edge_index, batch, W1, att_src1, att_dst1, b1,
           W2, att_src2, att_dst2, b2, linW, linb, outW, outb):
    loop = jnp.arange(N, dtype=edge_index.dtype)
    src = jnp.concatenate([edge_index[0], loop])
    dst = jnp.concatenate([edge_index[1], loop])
    pad = ETOT_PAD - ETOT
    zpad = jnp.zeros((pad,), dtype=src.dtype)
    src4 = jnp.concatenate([src, zpad]).reshape(NW, NPAIR, 2, CHUNK)
    dst4 = jnp.concatenate([dst, zpad]).reshape(NW, NPAIR, 2, CHUNK)
    src4 = jnp.pad(src4, ((0, 0), (0, 1), (0, 0), (0, 0)))
    dst4 = jnp.pad(dst4, ((0, 0), (0, 1), (0, 0), (0, 0)))
    z_rows = jnp.zeros((N_PAD, WD), jnp.float32)

    def aug_w(wmat, att_s):
        acol = wmat @ att_s.reshape(C, 1)
        zcol = jnp.zeros_like(acol)
        tail = jnp.zeros((wmat.shape[0], WD - C - 2), wmat.dtype)
        return jnp.concatenate([wmat, zcol, acol, tail], axis=1)

    W1p = aug_w(W1, att_src1)
    W2p = aug_w(W2, att_src2)
    attd1 = jnp.pad(att_dst1.reshape(1, C), ((0, 0), (0, WD - C)))
    attd2 = jnp.pad(att_dst2.reshape(1, C), ((0, 0), (0, WD - C)))
    batch2 = batch.reshape(N, 1)
    b1r = b1.reshape(1, C)
    b2r = b2.reshape(1, C)
    linbr = linb.reshape(1, LIN)
    outbr = outb.reshape(1, OUT)

    h1, ad1, ms1 = _tc_proj(x, W1p, attd1)
    p1 = _sc_edge_pass(h1, ad1, ms1, src4, dst4, z_rows)
    h2, ad2, ms2 = _tc_combine_proj(p1, b1r, W2p, attd2)
    p2 = _sc_edge_pass(h2, ad2, ms2, src4, dst4, z_rows)
    return _tc_head(p2, b2r, batch2, linW, linbr, outW, outbr)
